# baseline (speedup 1.0000x reference)
import functools

import jax
import jax.numpy as jnp
from jax import lax
from jax.experimental import pallas as pl
from jax.experimental.pallas import tpu as pltpu
from jax.experimental.pallas import tpu_sc as plsc

NUM_CORES = 2
NUM_SUBCORES = 16
NUM_WORKERS = NUM_CORES * NUM_SUBCORES

EMBED_DIM = 32
IRREPS_DIM = 128
CHUNK = 256
NBUF = 3
DENSE_BLK = 2000


def _fuse_body(tab_ref, w_ref, o_ref):
    o_ref[...] = jnp.dot(
        tab_ref[...], w_ref[...], preferred_element_type=jnp.float32
    ) / jnp.sqrt(jnp.float32(EMBED_DIM))


def _dense_body(idx_ref, tab_ref, o1_ref, o2_ref):
    idx = idx_ref[...]
    num_types = tab_ref.shape[0]
    onehot = (idx == lax.broadcasted_iota(
        jnp.int32, (1, num_types), 1)).astype(jnp.float32)
    d = jnp.dot(onehot, tab_ref[...], preferred_element_type=jnp.float32)
    o1_ref[...] = d
    o2_ref[...] = d


def _make_sc_gather(n):
    n_full = n // CHUNK
    tail = n - n_full * CHUNK
    tail_base = n_full * CHUNK
    tail_worker = NUM_WORKERS - 1
    cnt_lo = n_full // NUM_WORKERS
    extra = n_full - cnt_lo * NUM_WORKERS
    cnt_hi = cnt_lo + (1 if extra else 0)
    n_groups = -(-cnt_hi // NBUF)

    mesh = plsc.VectorSubcoreMesh(
        core_axis_name="c", subcore_axis_name="s",
        num_cores=NUM_CORES, num_subcores=NUM_SUBCORES,
    )

    scratch = [
        pltpu.VMEM((cnt_hi * CHUNK,), jnp.int32),
    ]
    scratch += [pltpu.VMEM((CHUNK, IRREPS_DIM), jnp.float32) for _ in range(NBUF)]
    scratch += [pltpu.SemaphoreType.DMA for _ in range(2 * NBUF)]

    @functools.partial(
        pl.kernel,
        out_type=jax.ShapeDtypeStruct((n, IRREPS_DIM), jnp.float32),
        mesh=mesh,
        scratch_types=scratch,
        compiler_params=pltpu.CompilerParams(use_tc_tiling_on_sc=False),
    )
    def sc_gather(idx_hbm, fused_hbm, ne_hbm, idx_all, *bufs_sems):
        bufs = bufs_sems[:NBUF]
        sems_g = bufs_sems[NBUF:2 * NBUF]
        sems_w = bufs_sems[2 * NBUF:]
        w = lax.axis_index("s") * NUM_CORES + lax.axis_index("c")
        cnt = jnp.where(w < extra, cnt_hi, cnt_lo)
        start = cnt_lo * w + jnp.minimum(w, extra)

        if extra:
            @pl.when(w < extra)
            def _():
                pltpu.sync_copy(
                    idx_hbm.at[pl.ds(start * CHUNK, cnt_hi * CHUNK)], idx_all)

            @pl.when(w >= extra)
            def _():
                pltpu.sync_copy(
                    idx_hbm.at[pl.ds(start * CHUNK, cnt_lo * CHUNK)],
                    idx_all.at[pl.ds(0, cnt_lo * CHUNK)])
        else:
            pltpu.sync_copy(
                idx_hbm.at[pl.ds(start * CHUNK, cnt_lo * CHUNK)], idx_all)

        def group_body(g, carry):
            copies_g = []
            for b in range(NBUF):
                s = g * NBUF + b

                @pl.when(s < cnt)
                def _(s=s, b=b):
                    pltpu.async_copy(
                        fused_hbm.at[idx_all.at[pl.ds(s * CHUNK, CHUNK)]],
                        bufs[b], sems_g[b])

            for b in range(NBUF):
                s = g * NBUF + b

                @pl.when(s < cnt)
                def _(s=s, b=b):
                    pltpu.make_async_copy(
                        fused_hbm.at[idx_all.at[pl.ds(s * CHUNK, CHUNK)]],
                        bufs[b], sems_g[b]).wait()
                    pltpu.async_copy(
                        bufs[b], ne_hbm.at[pl.ds((start + s) * CHUNK, CHUNK)],
                        sems_w[b])

            for b in range(NBUF):
                s = g * NBUF + b

                @pl.when(s < cnt)
                def _(s=s, b=b):
                    pltpu.make_async_copy(
                        bufs[b], ne_hbm.at[pl.ds((start + s) * CHUNK, CHUNK)],
                        sems_w[b]).wait()

            return carry

        lax.fori_loop(0, n_groups, group_body, 0)

        if tail:
            @pl.when(w == tail_worker)
            def _():
                pltpu.sync_copy(idx_hbm.at[pl.ds(tail_base, tail)],
                                idx_all.at[pl.ds(0, tail)])
                pltpu.async_copy(
                    fused_hbm.at[idx_all.at[pl.ds(0, tail)]],
                    bufs[0].at[pl.ds(0, tail)], sems_g[0]).wait()
                pltpu.sync_copy(bufs[0].at[pl.ds(0, tail)],
                                ne_hbm.at[pl.ds(tail_base, tail)])

    return sc_gather


def kernel(node_atom, embed_table, W):
    node_atom = node_atom.astype(jnp.int32)
    n = node_atom.shape[0]
    num_types = embed_table.shape[0]

    fused = pl.pallas_call(
        _fuse_body,
        out_shape=jax.ShapeDtypeStruct((num_types, IRREPS_DIM), jnp.float32),
    )(embed_table, W)

    node_embedding = _make_sc_gather(n)(node_atom, fused)

    blk = DENSE_BLK if n % DENSE_BLK == 0 else n
    grid = n // blk
    idx2d = node_atom.reshape(n, 1)
    atom_attr, atom_dense = pl.pallas_call(
        _dense_body,
        grid=(grid,),
        in_specs=[
            pl.BlockSpec((blk, 1), lambda i: (i, 0)),
            pl.BlockSpec((num_types, EMBED_DIM), lambda i: (0, 0)),
        ],
        out_specs=[
            pl.BlockSpec((blk, EMBED_DIM), lambda i: (i, 0)),
            pl.BlockSpec((blk, EMBED_DIM), lambda i: (i, 0)),
        ],
        out_shape=[
            jax.ShapeDtypeStruct((n, EMBED_DIM), jnp.float32),
            jax.ShapeDtypeStruct((n, EMBED_DIM), jnp.float32),
        ],
    )(idx2d, embed_table)

    return (node_embedding, atom_attr, atom_dense)


# speedup vs baseline: 1.1687x; 1.1687x over previous
import functools

import jax
import jax.numpy as jnp
from jax import lax
from jax.experimental import pallas as pl
from jax.experimental.pallas import tpu as pltpu
from jax.experimental.pallas import tpu_sc as plsc

NUM_CORES = 2
NUM_SUBCORES = 16
NUM_WORKERS = NUM_CORES * NUM_SUBCORES

EMBED_DIM = 32
IRREPS_DIM = 128
CHUNK = 256
NBUF = 3
DENSE_BLK = 10000


def _fuse_body(tab_ref, w_ref, o_ref):
    o_ref[...] = jnp.dot(
        tab_ref[...], w_ref[...], preferred_element_type=jnp.float32
    ) / jnp.sqrt(jnp.float32(EMBED_DIM))


def _dense_body(idx_ref, tab_ref, o1_ref, o2_ref):
    idx = idx_ref[0]
    num_types = tab_ref.shape[0]
    onehot = (idx == lax.broadcasted_iota(
        jnp.int32, (num_types, 1), 0)).astype(jnp.float32)
    d = lax.dot_general(
        onehot, tab_ref[...], (((0,), (0,)), ((), ())),
        preferred_element_type=jnp.float32)
    o1_ref[...] = d
    o2_ref[...] = d


def _make_sc_gather(n):
    n_full = n // CHUNK
    tail = n - n_full * CHUNK
    tail_base = n_full * CHUNK
    tail_worker = NUM_WORKERS - 1
    cnt_lo = n_full // NUM_WORKERS
    extra = n_full - cnt_lo * NUM_WORKERS
    cnt_hi = cnt_lo + (1 if extra else 0)
    n_groups = -(-cnt_hi // NBUF)

    mesh = plsc.VectorSubcoreMesh(
        core_axis_name="c", subcore_axis_name="s",
        num_cores=NUM_CORES, num_subcores=NUM_SUBCORES,
    )

    scratch = [
        pltpu.VMEM((cnt_hi * CHUNK,), jnp.int32),
    ]
    scratch += [pltpu.VMEM((CHUNK, IRREPS_DIM), jnp.float32) for _ in range(NBUF)]
    scratch += [pltpu.SemaphoreType.DMA for _ in range(2 * NBUF)]

    @functools.partial(
        pl.kernel,
        out_type=jax.ShapeDtypeStruct((n, IRREPS_DIM), jnp.float32),
        mesh=mesh,
        scratch_types=scratch,
        compiler_params=pltpu.CompilerParams(use_tc_tiling_on_sc=False),
    )
    def sc_gather(idx_hbm, fused_hbm, ne_hbm, idx_all, *bufs_sems):
        bufs = bufs_sems[:NBUF]
        sems_g = bufs_sems[NBUF:2 * NBUF]
        sems_w = bufs_sems[2 * NBUF:]
        w = lax.axis_index("s") * NUM_CORES + lax.axis_index("c")
        cnt = jnp.where(w < extra, cnt_hi, cnt_lo)
        start = cnt_lo * w + jnp.minimum(w, extra)

        if extra:
            @pl.when(w < extra)
            def _():
                pltpu.sync_copy(
                    idx_hbm.at[pl.ds(start * CHUNK, cnt_hi * CHUNK)], idx_all)

            @pl.when(w >= extra)
            def _():
                pltpu.sync_copy(
                    idx_hbm.at[pl.ds(start * CHUNK, cnt_lo * CHUNK)],
                    idx_all.at[pl.ds(0, cnt_lo * CHUNK)])
        else:
            pltpu.sync_copy(
                idx_hbm.at[pl.ds(start * CHUNK, cnt_lo * CHUNK)], idx_all)

        def group_body(g, carry):
            copies_g = []
            for b in range(NBUF):
                s = g * NBUF + b

                @pl.when(s < cnt)
                def _(s=s, b=b):
                    pltpu.async_copy(
                        fused_hbm.at[idx_all.at[pl.ds(s * CHUNK, CHUNK)]],
                        bufs[b], sems_g[b])

            for b in range(NBUF):
                s = g * NBUF + b

                @pl.when(s < cnt)
                def _(s=s, b=b):
                    pltpu.make_async_copy(
                        fused_hbm.at[idx_all.at[pl.ds(s * CHUNK, CHUNK)]],
                        bufs[b], sems_g[b]).wait()
                    pltpu.async_copy(
                        bufs[b], ne_hbm.at[pl.ds((start + s) * CHUNK, CHUNK)],
                        sems_w[b])

            for b in range(NBUF):
                s = g * NBUF + b

                @pl.when(s < cnt)
                def _(s=s, b=b):
                    pltpu.make_async_copy(
                        bufs[b], ne_hbm.at[pl.ds((start + s) * CHUNK, CHUNK)],
                        sems_w[b]).wait()

            return carry

        lax.fori_loop(0, n_groups, group_body, 0)

        if tail:
            @pl.when(w == tail_worker)
            def _():
                pltpu.sync_copy(idx_hbm.at[pl.ds(tail_base, tail)],
                                idx_all.at[pl.ds(0, tail)])
                pltpu.async_copy(
                    fused_hbm.at[idx_all.at[pl.ds(0, tail)]],
                    bufs[0].at[pl.ds(0, tail)], sems_g[0]).wait()
                pltpu.sync_copy(bufs[0].at[pl.ds(0, tail)],
                                ne_hbm.at[pl.ds(tail_base, tail)])

    return sc_gather


def kernel(node_atom, embed_table, W):
    node_atom = node_atom.astype(jnp.int32)
    n = node_atom.shape[0]
    num_types = embed_table.shape[0]

    fused = pl.pallas_call(
        _fuse_body,
        out_shape=jax.ShapeDtypeStruct((num_types, IRREPS_DIM), jnp.float32),
    )(embed_table, W)

    node_embedding = _make_sc_gather(n)(node_atom, fused)

    blk = DENSE_BLK if n % DENSE_BLK == 0 else n
    grid = n // blk
    idx3d = node_atom.reshape(grid, 1, blk)
    atom_attr, atom_dense = pl.pallas_call(
        _dense_body,
        grid=(grid,),
        in_specs=[
            pl.BlockSpec((1, 1, blk), lambda i: (i, 0, 0)),
            pl.BlockSpec((num_types, EMBED_DIM), lambda i: (0, 0)),
        ],
        compiler_params=pltpu.CompilerParams(
            fuse_transposed_lhs_in_matmul=True),
        out_specs=[
            pl.BlockSpec((blk, EMBED_DIM), lambda i: (i, 0)),
            pl.BlockSpec((blk, EMBED_DIM), lambda i: (i, 0)),
        ],
        out_shape=[
            jax.ShapeDtypeStruct((n, EMBED_DIM), jnp.float32),
            jax.ShapeDtypeStruct((n, EMBED_DIM), jnp.float32),
        ],
    )(idx3d, embed_table)

    return (node_embedding, atom_attr, atom_dense)


# speedup vs baseline: 1.9938x; 1.7060x over previous
import functools

import jax
import jax.numpy as jnp
from jax import lax
from jax.experimental import pallas as pl
from jax.experimental.pallas import tpu as pltpu
from jax.experimental.pallas import tpu_sc as plsc

NUM_CORES = 2
NUM_SUBCORES = 16
NUM_WORKERS = NUM_CORES * NUM_SUBCORES

EMBED_DIM = 32
IRREPS_DIM = 128
CHUNK = 256
NBUF = 2
DENSE_BLK = 10000


def _fuse_body(tab_ref, w_ref, o_ref):
    o_ref[...] = jnp.dot(
        tab_ref[...], w_ref[...], preferred_element_type=jnp.float32
    ) / jnp.sqrt(jnp.float32(EMBED_DIM))


def _dense_body(idx_ref, tab_ref, o1_ref, o2_ref):
    idx = idx_ref[0]
    num_types = tab_ref.shape[0]
    onehot = (idx == lax.broadcasted_iota(
        jnp.int32, (num_types, 1), 0)).astype(jnp.float32)
    d = lax.dot_general(
        onehot, tab_ref[...], (((0,), (0,)), ((), ())),
        preferred_element_type=jnp.float32)
    o1_ref[...] = d
    o2_ref[...] = d


def _make_sc_gather(n):
    n_full = n // CHUNK
    tail = n - n_full * CHUNK
    tail_base = n_full * CHUNK
    tail_worker = NUM_WORKERS - 1
    cnt_lo = n_full // NUM_WORKERS
    extra = n_full - cnt_lo * NUM_WORKERS
    cnt_hi = cnt_lo + (1 if extra else 0)
    n_slots = -(-cnt_hi // NBUF) * NBUF
    n_groups = n_slots // NBUF

    mesh = plsc.VectorSubcoreMesh(
        core_axis_name="c", subcore_axis_name="s",
        num_cores=NUM_CORES, num_subcores=NUM_SUBCORES,
    )

    scratch = [
        pltpu.VMEM((cnt_hi * CHUNK,), jnp.int32),
        pltpu.VMEM((64, IRREPS_DIM), jnp.float32),
    ]
    scratch += [pltpu.VMEM((CHUNK, IRREPS_DIM), jnp.float32) for _ in range(NBUF)]
    scratch += [pltpu.SemaphoreType.DMA for _ in range(NBUF)]

    @functools.partial(
        pl.kernel,
        out_type=jax.ShapeDtypeStruct((n, IRREPS_DIM), jnp.float32),
        mesh=mesh,
        scratch_types=scratch,
        compiler_params=pltpu.CompilerParams(use_tc_tiling_on_sc=False),
    )
    def sc_gather(idx_hbm, fused_hbm, ne_hbm, idx_all, fused_v, *bufs_sems):
        bufs = bufs_sems[:NBUF]
        sems_w = bufs_sems[NBUF:]
        w = lax.axis_index("s") * NUM_CORES + lax.axis_index("c")
        cnt = jnp.where(w < extra, cnt_hi, cnt_lo)
        start = cnt_lo * w + jnp.minimum(w, extra)

        pltpu.sync_copy(fused_hbm, fused_v)
        if extra:
            @pl.when(w < extra)
            def _():
                pltpu.sync_copy(
                    idx_hbm.at[pl.ds(start * CHUNK, cnt_hi * CHUNK)], idx_all)

            @pl.when(w >= extra)
            def _():
                pltpu.sync_copy(
                    idx_hbm.at[pl.ds(start * CHUNK, cnt_lo * CHUNK)],
                    idx_all.at[pl.ds(0, cnt_lo * CHUNK)])
        else:
            pltpu.sync_copy(
                idx_hbm.at[pl.ds(start * CHUNK, cnt_lo * CHUNK)], idx_all)

        def assemble_rows(ibase, buf, nrows):
            def grp_body(k, carry):
                iv = idx_all[pl.ds(ibase + k * 16, 16)]
                for j in range(16):
                    t = iv[j]
                    r = k * 16 + j
                    for q in range(IRREPS_DIM // 16):
                        buf[r, pl.ds(q * 16, 16)] = fused_v[t, pl.ds(q * 16, 16)]
                return carry

            lax.fori_loop(0, nrows // 16, grp_body, 0)

        def group_body(g, carry):
            for b in range(NBUF):
                s = g * NBUF + b
                s_eff = jnp.minimum(s, cnt - 1)

                @pl.when(g > 0)
                def _(b=b):
                    pltpu.make_async_copy(
                        bufs[b], ne_hbm.at[pl.ds(0, CHUNK)], sems_w[b]).wait()

                assemble_rows(s_eff * CHUNK, bufs[b], CHUNK)
                pltpu.async_copy(
                    bufs[b], ne_hbm.at[pl.ds((start + s_eff) * CHUNK, CHUNK)],
                    sems_w[b])
            return carry

        lax.fori_loop(0, n_groups, group_body, 0)
        for b in range(NBUF):
            pltpu.make_async_copy(
                bufs[b], ne_hbm.at[pl.ds(0, CHUNK)], sems_w[b]).wait()

        if tail:
            @pl.when(w == tail_worker)
            def _():
                pltpu.sync_copy(idx_hbm.at[pl.ds(tail_base, tail)],
                                idx_all.at[pl.ds(0, tail)])
                assemble_rows(0, bufs[0], tail)
                pltpu.sync_copy(bufs[0].at[pl.ds(0, tail)],
                                ne_hbm.at[pl.ds(tail_base, tail)])

    return sc_gather


def kernel(node_atom, embed_table, W):
    node_atom = node_atom.astype(jnp.int32)
    n = node_atom.shape[0]
    num_types = embed_table.shape[0]

    fused = pl.pallas_call(
        _fuse_body,
        out_shape=jax.ShapeDtypeStruct((num_types, IRREPS_DIM), jnp.float32),
    )(embed_table, W)

    node_embedding = _make_sc_gather(n)(node_atom, fused)

    blk = DENSE_BLK if n % DENSE_BLK == 0 else n
    grid = n // blk
    idx3d = node_atom.reshape(grid, 1, blk)
    atom_attr, atom_dense = pl.pallas_call(
        _dense_body,
        grid=(grid,),
        in_specs=[
            pl.BlockSpec((1, 1, blk), lambda i: (i, 0, 0)),
            pl.BlockSpec((num_types, EMBED_DIM), lambda i: (0, 0)),
        ],
        compiler_params=pltpu.CompilerParams(
            fuse_transposed_lhs_in_matmul=True),
        out_specs=[
            pl.BlockSpec((blk, EMBED_DIM), lambda i: (i, 0)),
            pl.BlockSpec((blk, EMBED_DIM), lambda i: (i, 0)),
        ],
        out_shape=[
            jax.ShapeDtypeStruct((n, EMBED_DIM), jnp.float32),
            jax.ShapeDtypeStruct((n, EMBED_DIM), jnp.float32),
        ],
    )(idx3d, embed_table)

    return (node_embedding, atom_attr, atom_dense)
